# baseline (device time: 8362 ns/iter reference)
import jax
import jax.numpy as jnp
from jax import lax
from jax.experimental import pallas as pl
from jax.experimental.pallas import tpu as pltpu

NX, NY, NZ = 2, 2, 4
S = 48

XM, XP, YM, YP, ZM, ZP = 0, 1, 2, 3, 4, 5


def kernel(u):
    def body(u_ref, out_ref, stage_ref, halo_ref, send_sems, recv_sems):
        px = lax.axis_index("x")
        py = lax.axis_index("y")
        pz = lax.axis_index("z")

        has = {
            XM: px > 0,
            XP: px < NX - 1,
            YM: py > 0,
            YP: py < NY - 1,
            ZM: pz > 0,
            ZP: pz < NZ - 1,
        }
        nbr = {
            XM: (px - 1, py, pz),
            XP: (px + 1, py, pz),
            YM: (px, py - 1, pz),
            YP: (px, py + 1, pz),
            ZM: (px, py, pz - 1),
            ZP: (px, py, pz + 1),
        }
        opposite = {XM: XP, XP: XM, YM: YP, YP: YM, ZM: ZP, ZP: ZM}

        barrier = pltpu.get_barrier_semaphore()
        for d in range(6):
            @pl.when(has[d])
            def _(d=d):
                pl.semaphore_signal(
                    barrier, inc=1,
                    device_id=nbr[d], device_id_type=pl.DeviceIdType.MESH,
                )

        u_val = u_ref[...]
        stage_ref[YM] = u_val[:, 0, :]
        stage_ref[YP] = u_val[:, S - 1, :]
        stage_ref[ZM] = u_val[:, :, 0]
        stage_ref[ZP] = u_val[:, :, S - 1]

        zpad_x = jnp.zeros((1, S, S), jnp.float32)
        zpad_y = jnp.zeros((S, 1, S), jnp.float32)
        um_x = jnp.concatenate([zpad_x, u_val[:-1]], axis=0)
        up_x = jnp.concatenate([u_val[1:], zpad_x], axis=0)
        um_y = jnp.concatenate([zpad_y, u_val[:, :-1]], axis=1)
        up_y = jnp.concatenate([u_val[:, 1:], zpad_y], axis=1)
        v_xy = um_x + up_x + um_y + up_y - 6.0 * u_val

        n_nbrs = (
            2
            + (pz > 0).astype(jnp.int32)
            + (pz < NZ - 1).astype(jnp.int32)
        )
        pl.semaphore_wait(barrier, n_nbrs)

        src_for = {
            XM: u_ref.at[0],
            XP: u_ref.at[S - 1],
            YM: stage_ref.at[YM],
            YP: stage_ref.at[YP],
            ZM: stage_ref.at[ZM],
            ZP: stage_ref.at[ZP],
        }

        def rdma_for(d):
            return pltpu.make_async_remote_copy(
                src_ref=src_for[d],
                dst_ref=halo_ref.at[opposite[d]],
                send_sem=send_sems.at[d],
                recv_sem=recv_sems.at[opposite[d]],
                device_id=nbr[d],
                device_id_type=pl.DeviceIdType.MESH,
            )

        for d in range(6):
            @pl.when(has[d])
            def _(d=d):
                rdma_for(d).start()

        zpad_z = jnp.zeros((S, S, 1), jnp.float32)
        um_z = jnp.concatenate([zpad_z, u_val[:, :, :-1]], axis=2)
        up_z = jnp.concatenate([u_val[:, :, 1:], zpad_z], axis=2)
        v = v_xy + um_z + up_z

        def span_mask(shape, da, lo_a, hi_a, db, lo_b, hi_b):
            a = lax.broadcasted_iota(jnp.int32, shape, da)
            b = lax.broadcasted_iota(jnp.int32, shape, db)
            return (
                ((a > 0) | lo_a) & ((a < S - 1) | hi_a)
                & ((b > 0) | lo_b) & ((b < S - 1) | hi_b)
            )

        mask_x = span_mask((1, S, S), 1, has[YM], has[YP], 2, has[ZM], has[ZP])
        mask_y = span_mask((S, 1, S), 0, has[XM], has[XP], 2, has[ZM], has[ZP])
        mask_z = span_mask((S, S, 1), 0, has[XM], has[XP], 1, has[YM], has[YP])
        face_mask = {XM: mask_x, XP: mask_x, YM: mask_y, YP: mask_y,
                     ZM: mask_z, ZP: mask_z}

        for d in range(6):
            @pl.when(has[d])
            def _(d=d):
                pltpu.make_async_remote_copy(
                    src_ref=src_for[d],
                    dst_ref=halo_ref.at[d],
                    send_sem=send_sems.at[d],
                    recv_sem=recv_sems.at[d],
                    device_id=nbr[d],
                    device_id_type=pl.DeviceIdType.MESH,
                ).wait_recv()

        def plane(d, vp, hp):
            return jnp.where(face_mask[d] & has[d], vp + hp, 0.0)

        v = jnp.concatenate(
            [plane(XM, v[0:1], halo_ref[XM][None]),
             v[1:S - 1],
             plane(XP, v[S - 1:], halo_ref[XP][None])],
            axis=0,
        )
        v = jnp.concatenate(
            [plane(YM, v[:, 0:1], halo_ref[YM][:, None]),
             v[:, 1:S - 1],
             plane(YP, v[:, S - 1:], halo_ref[YP][:, None])],
            axis=1,
        )
        v = jnp.concatenate(
            [plane(ZM, v[:, :, 0:1], halo_ref[ZM][:, :, None]),
             v[:, :, 1:S - 1],
             plane(ZP, v[:, :, S - 1:], halo_ref[ZP][:, :, None])],
            axis=2,
        )
        out_ref[...] = v

        for d in range(6):
            @pl.when(has[d])
            def _(d=d):
                rdma_for(d).wait_send()

    return pl.pallas_call(
        body,
        out_shape=jax.ShapeDtypeStruct((S, S, S), jnp.float32),
        in_specs=[pl.BlockSpec(memory_space=pltpu.VMEM)],
        out_specs=pl.BlockSpec(memory_space=pltpu.VMEM),
        scratch_shapes=[
            pltpu.VMEM((6, S, S), jnp.float32),
            pltpu.VMEM((6, S, S), jnp.float32),
            pltpu.SemaphoreType.DMA((6,)),
            pltpu.SemaphoreType.DMA((6,)),
        ],
        compiler_params=pltpu.CompilerParams(collective_id=0),
    )(u)
